# baseline (speedup 1.0000x reference)
import jax
import jax.numpy as jnp
from jax.experimental import pallas as pl
from jax.experimental.pallas import tpu as pltpu

B = 32
C = 768
EPS = 1e-10
NBUF = 4
NCH = 4
CCH = C // NCH


def _kernel(x_hbm, quant_hbm, ind_ref, div_ref, perp_ref,
            in_buf, out_buf, qacc, cacc, in_sem, out_sem):
    b = pl.program_id(0)
    slot = jax.lax.rem(b, NBUF)
    xv = x_hbm
    qv = quant_hbm

    def in_copy(bb, s, j):
        return pltpu.make_async_copy(
            xv.at[bb, pl.ds(j * CCH, CCH)],
            in_buf.at[s, pl.ds(j * CCH, CCH)],
            in_sem.at[s, j])

    def out_copy(bb, s, j):
        return pltpu.make_async_copy(
            out_buf.at[s, pl.ds(j * CCH, CCH)],
            qv.at[bb, pl.ds(j * CCH, CCH)],
            out_sem.at[s, j])

    @pl.when(b == 0)
    def _warmup():
        for k in range(NBUF - 1):
            for j in range(NCH):
                in_copy(k, k, j).start()
        qacc[...] = jnp.zeros_like(qacc)
        cacc[...] = jnp.zeros_like(cacc)

    @pl.when(b + NBUF - 1 < B)
    def _prefetch():
        for j in range(NCH):
            in_copy(b + NBUF - 1, jax.lax.rem(b + NBUF - 1, NBUF), j).start()

    for j in range(NCH):
        in_copy(b, slot, j).wait()

    @pl.when(b >= NBUF)
    def _reclaim():
        for j in range(NCH):
            out_copy(b - NBUF, slot, j).wait()

    x = in_buf[slot]
    r = jnp.maximum(x, 0.0)
    s = jnp.sum(r, axis=0)
    w = 1.0 / (s + EPS)

    m = jnp.max(r, axis=0)
    ci = jax.lax.broadcasted_iota(jnp.int32, (C, 8, 128), 0)
    idx = jnp.min(jnp.where(r == m[None], ci, C), axis=0)

    onehot = (ci == idx[None]).astype(jnp.float32)
    out_buf[slot] = onehot
    ind_ref[0] = idx

    qacc[...] += r * w[None]
    cacc[...] += onehot

    for j in range(NCH):
        out_copy(b, slot, j).start()

    @pl.when(b == B - 1)
    def _drain():
        for k in range(NBUF):
            bb = B - NBUF + k
            for j in range(NCH):
                out_copy(bb, jax.lax.rem(bb, NBUF), j).wait()
        n = float(B * 1024)
        qs = jnp.sum(jnp.sum(qacc[...], axis=1), axis=1, keepdims=True)
        q_bar = qs / n
        div_ref[...] = jnp.mean((q_bar * C - 1.0) ** 2, keepdims=True)
        p = jnp.sum(jnp.sum(cacc[...], axis=1), axis=1, keepdims=True) / n
        perp_ref[...] = jnp.exp(-jnp.sum(p * jnp.log(p + 1e-10), keepdims=True))


@jax.jit
def kernel(x):
    xr = x.reshape(B, C, 8, 128)
    quant, ind, div, perp = pl.pallas_call(
        _kernel,
        grid=(B,),
        in_specs=[pl.BlockSpec(memory_space=pltpu.MemorySpace.HBM)],
        out_specs=[
            pl.BlockSpec(memory_space=pltpu.MemorySpace.HBM),
            pl.BlockSpec((1, 8, 128), lambda b: (b, 0, 0)),
            pl.BlockSpec((1, 1), lambda b: (0, 0)),
            pl.BlockSpec((1, 1), lambda b: (0, 0)),
        ],
        out_shape=[
            jax.ShapeDtypeStruct((B, C, 8, 128), jnp.float32),
            jax.ShapeDtypeStruct((B, 8, 128), jnp.int32),
            jax.ShapeDtypeStruct((1, 1), jnp.float32),
            jax.ShapeDtypeStruct((1, 1), jnp.float32),
        ],
        scratch_shapes=[
            pltpu.VMEM((NBUF, C, 8, 128), jnp.float32),
            pltpu.VMEM((NBUF, C, 8, 128), jnp.float32),
            pltpu.VMEM((C, 8, 128), jnp.float32),
            pltpu.VMEM((C, 8, 128), jnp.float32),
            pltpu.SemaphoreType.DMA((NBUF, NCH)),
            pltpu.SemaphoreType.DMA((NBUF, NCH)),
        ],
    )(xr)
    quantize = quant.reshape(B, C, 32, 32)
    embed_ind = ind.reshape(B, 32, 32)
    return (quantize, div[0, 0], embed_ind, perp[0, 0])


# speedup vs baseline: 1.0747x; 1.0747x over previous
import jax
import jax.numpy as jnp
from jax.experimental import pallas as pl
from jax.experimental.pallas import tpu as pltpu

B = 32
C = 768
HW = 1024
EPS = 1e-10
NBUF = 4
NCH = 4
CCH = C // NCH


def _kernel(x_hbm, quant_hbm, ind_ref, div_ref, perp_ref,
            in_buf, out_buf, qsum_ref, cnt_ref, in_sem, out_sem):
    b = pl.program_id(0)
    slot = jax.lax.rem(b, NBUF)

    def in_copy(bb, s, j):
        return pltpu.make_async_copy(
            x_hbm.at[bb, pl.ds(j * CCH, CCH)],
            in_buf.at[s, pl.ds(j * CCH, CCH)],
            in_sem.at[s, j])

    def out_copy(bb, s, j):
        return pltpu.make_async_copy(
            out_buf.at[s, pl.ds(j * CCH, CCH)],
            quant_hbm.at[bb, pl.ds(j * CCH, CCH)],
            out_sem.at[s, j])

    @pl.when(b == 0)
    def _warmup():
        for k in range(NBUF - 1):
            for j in range(NCH):
                in_copy(k, k, j).start()
        qsum_ref[...] = jnp.zeros_like(qsum_ref)
        cnt_ref[...] = jnp.zeros_like(cnt_ref)

    @pl.when(b + NBUF - 1 < B)
    def _prefetch():
        for j in range(NCH):
            in_copy(b + NBUF - 1, jax.lax.rem(b + NBUF - 1, NBUF), j).start()

    for j in range(NCH):
        in_copy(b, slot, j).wait()

    @pl.when(b >= NBUF)
    def _reclaim():
        for j in range(NCH):
            out_copy(b - NBUF, slot, j).wait()

    x = in_buf[slot]
    r = jnp.maximum(x, 0.0)
    s = jnp.sum(r, axis=0, keepdims=True)
    w = 1.0 / (s + EPS)

    m = jnp.max(r, axis=0, keepdims=True)
    ci = jax.lax.broadcasted_iota(jnp.int32, (C, HW), 0)
    idx = jnp.min(jnp.where(r == m, ci, C), axis=0, keepdims=True)

    onehot = (ci == idx).astype(jnp.float32)
    out_buf[slot] = onehot
    ind_ref[0] = idx

    qsum_ref[...] += jnp.sum(r * w, axis=1, keepdims=True)
    cnt_ref[...] += jnp.sum(onehot, axis=1, keepdims=True)

    for j in range(NCH):
        out_copy(b, slot, j).start()

    @pl.when(b == B - 1)
    def _drain():
        for k in range(NBUF):
            bb = B - NBUF + k
            for j in range(NCH):
                out_copy(bb, jax.lax.rem(bb, NBUF), j).wait()
        n = float(B * HW)
        q_bar = qsum_ref[...] / n
        div_ref[...] = jnp.mean((q_bar * C - 1.0) ** 2, keepdims=True)
        p = cnt_ref[...] / n
        perp_ref[...] = jnp.exp(-jnp.sum(p * jnp.log(p + 1e-10), keepdims=True))


@jax.jit
def kernel(x):
    xr = x.reshape(B, C, HW)
    quant, ind, div, perp = pl.pallas_call(
        _kernel,
        grid=(B,),
        in_specs=[pl.BlockSpec(memory_space=pltpu.MemorySpace.HBM)],
        out_specs=[
            pl.BlockSpec(memory_space=pltpu.MemorySpace.HBM),
            pl.BlockSpec((1, 1, HW), lambda b: (b, 0, 0)),
            pl.BlockSpec((1, 1), lambda b: (0, 0)),
            pl.BlockSpec((1, 1), lambda b: (0, 0)),
        ],
        out_shape=[
            jax.ShapeDtypeStruct((B, C, HW), jnp.float32),
            jax.ShapeDtypeStruct((B, 1, HW), jnp.int32),
            jax.ShapeDtypeStruct((1, 1), jnp.float32),
            jax.ShapeDtypeStruct((1, 1), jnp.float32),
        ],
        scratch_shapes=[
            pltpu.VMEM((NBUF, C, HW), jnp.float32),
            pltpu.VMEM((NBUF, C, HW), jnp.float32),
            pltpu.VMEM((C, 1), jnp.float32),
            pltpu.VMEM((C, 1), jnp.float32),
            pltpu.SemaphoreType.DMA((NBUF, NCH)),
            pltpu.SemaphoreType.DMA((NBUF, NCH)),
        ],
    )(xr)
    quantize = quant.reshape(B, C, 32, 32)
    embed_ind = ind.reshape(B, 32, 32)
    return (quantize, div[0, 0], embed_ind, perp[0, 0])
